# baseline (speedup 1.0000x reference)
import functools

import jax
import jax.numpy as jnp
from jax import lax
from jax.experimental import pallas as pl
from jax.experimental.pallas import tpu as pltpu

E = 8
TOP_K = 2
D = 1024
DFF = 512
SHARED_DFF = 1024
T = 2048
TB = 256
NTB = T // TB

_PREC = lax.Precision.HIGHEST


def _top2_weights(x_blk, gate_w, e):
    logits = lax.dot_general(
        x_blk, gate_w, (((1,), (1,)), ((), ())),
        preferred_element_type=jnp.float32,
        precision=lax.Precision.DEFAULT)
    iota = lax.broadcasted_iota(jnp.int32, (x_blk.shape[0], E), 1)
    m1 = jnp.max(logits, axis=1, keepdims=True)
    a1 = jnp.min(jnp.where(logits == m1, iota, E), axis=1, keepdims=True)
    l2 = jnp.where(iota == a1, -jnp.inf, logits)
    m2 = jnp.max(l2, axis=1, keepdims=True)
    a2 = jnp.min(jnp.where(l2 == m2, iota, E), axis=1, keepdims=True)
    r = jnp.exp(m2 - m1)
    w1 = 1.0 / (1.0 + r)
    w2 = r / (1.0 + r)
    w_e = (jnp.where(a1 == e, w1, 0.0) + jnp.where(a2 == e, w2, 0.0))
    return w_e[:, 0]


def _glu(x, wg, wu, wd):
    g = lax.dot_general(x, wg, (((1,), (1,)), ((), ())),
                        preferred_element_type=jnp.float32, precision=_PREC)
    u = lax.dot_general(x, wu, (((1,), (1,)), ((), ())),
                        preferred_element_type=jnp.float32, precision=_PREC)
    h = (g * (1.0 / (1.0 + jnp.exp(-g)))) * u
    return lax.dot_general(h, wd, (((1,), (1,)), ((), ())),
                           preferred_element_type=jnp.float32, precision=_PREC)


def _moe_kernel(x_ref, gw_ref, wg_ref, wu_ref, wd_ref, swg_ref, swu_ref,
                swd_ref, out_ref, acc_ref):
    e = pl.program_id(0)
    tb = pl.program_id(1)
    x = x_ref[...]

    @pl.when(e < E)
    def _routed():
        w_e = _top2_weights(x, gw_ref[...], e)
        y = _glu(x, wg_ref[0], wu_ref[0], wd_ref[0])
        contrib = w_e[:, None] * y
        @pl.when(e == 0)
        def _():
            acc_ref[pl.ds(tb * TB, TB), :] = contrib
        @pl.when(e > 0)
        def _():
            acc_ref[pl.ds(tb * TB, TB), :] += contrib

    @pl.when(e == E)
    def _shared():
        y = _glu(x, swg_ref[...], swu_ref[...], swd_ref[...])
        acc_ref[pl.ds(tb * TB, TB), :] += y

    out_ref[...] = acc_ref[pl.ds(tb * TB, TB), :]


def kernel(hidden_states, gate_weight, Wg, Wu, Wd, sWg, sWu, sWd):
    grid = (E + 1, NTB)
    return pl.pallas_call(
        _moe_kernel,
        grid=grid,
        in_specs=[
            pl.BlockSpec((TB, D), lambda e, tb: (tb, 0)),
            pl.BlockSpec((E, D), lambda e, tb: (0, 0)),
            pl.BlockSpec((1, DFF, D), lambda e, tb: (jnp.minimum(e, E - 1), 0, 0)),
            pl.BlockSpec((1, DFF, D), lambda e, tb: (jnp.minimum(e, E - 1), 0, 0)),
            pl.BlockSpec((1, D, DFF), lambda e, tb: (jnp.minimum(e, E - 1), 0, 0)),
            pl.BlockSpec((SHARED_DFF, D), lambda e, tb: (0, 0)),
            pl.BlockSpec((SHARED_DFF, D), lambda e, tb: (0, 0)),
            pl.BlockSpec((D, SHARED_DFF), lambda e, tb: (0, 0)),
        ],
        out_specs=pl.BlockSpec((TB, D), lambda e, tb: (tb, 0)),
        out_shape=jax.ShapeDtypeStruct((T, D), jnp.float32),
        scratch_shapes=[pltpu.VMEM((T, D), jnp.float32)],
        compiler_params=pltpu.CompilerParams(
            dimension_semantics=("arbitrary", "arbitrary")),
    )(hidden_states, gate_weight, Wg, Wu, Wd, sWg, sWu, sWd)


# speedup vs baseline: 3.1417x; 3.1417x over previous
import functools

import jax
import jax.numpy as jnp
from jax import lax
from jax.experimental import pallas as pl
from jax.experimental.pallas import tpu as pltpu

E = 8
TOP_K = 2
D = 1024
DFF = 512
SHARED_DFF = 1024
T = 2048
TB = 256
NTB = T // TB

_PREC = lax.Precision.DEFAULT


def _top2_all_weights(x_blk, gate_w):
    logits = lax.dot_general(
        x_blk, gate_w, (((1,), (1,)), ((), ())),
        preferred_element_type=jnp.float32,
        precision=lax.Precision.DEFAULT)
    iota = lax.broadcasted_iota(jnp.int32, (x_blk.shape[0], E), 1)
    m1 = jnp.max(logits, axis=1, keepdims=True)
    a1 = jnp.min(jnp.where(logits == m1, iota, E), axis=1, keepdims=True)
    l2 = jnp.where(iota == a1, -jnp.inf, logits)
    m2 = jnp.max(l2, axis=1, keepdims=True)
    a2 = jnp.min(jnp.where(l2 == m2, iota, E), axis=1, keepdims=True)
    r = jnp.exp(m2 - m1)
    w1 = 1.0 / (1.0 + r)
    w2 = r / (1.0 + r)
    return jnp.where(iota == a1, w1, 0.0) + jnp.where(iota == a2, w2, 0.0)


def _glu(x, wg, wu, wd):
    g = lax.dot_general(x, wg, (((1,), (1,)), ((), ())),
                        preferred_element_type=jnp.float32, precision=_PREC)
    u = lax.dot_general(x, wu, (((1,), (1,)), ((), ())),
                        preferred_element_type=jnp.float32, precision=_PREC)
    h = (g * (1.0 / (1.0 + jnp.exp(-g)))) * u
    return lax.dot_general(h, wd, (((1,), (1,)), ((), ())),
                           preferred_element_type=jnp.float32, precision=_PREC)


def _moe_kernel(x_ref, gw_ref, wg_ref, wu_ref, wd_ref, swg_ref, swu_ref,
                swd_ref, out_ref, acc_ref, w_all_ref):
    e = pl.program_id(0)
    tb = pl.program_id(1)
    x = x_ref[...]

    @pl.when(e == 0)
    def _gate():
        w_all_ref[pl.ds(tb * TB, TB), :] = _top2_all_weights(x, gw_ref[...])

    @pl.when(e < E)
    def _routed():
        w_all = w_all_ref[pl.ds(tb * TB, TB), :]
        iota = lax.broadcasted_iota(jnp.int32, w_all.shape, 1)
        w_e = jnp.sum(jnp.where(iota == e, w_all, 0.0), axis=1)[:, None]
        y = _glu(x, wg_ref[0], wu_ref[0], wd_ref[0])
        contrib = w_e * y
        @pl.when(e == 0)
        def _():
            acc_ref[pl.ds(tb * TB, TB), :] = contrib
        @pl.when(e > 0)
        def _():
            acc_ref[pl.ds(tb * TB, TB), :] += contrib

    @pl.when(e == E)
    def _shared():
        y = _glu(x, swg_ref[...], swu_ref[...], swd_ref[...])
        acc_ref[pl.ds(tb * TB, TB), :] += y

    out_ref[...] = acc_ref[pl.ds(tb * TB, TB), :]


def kernel(hidden_states, gate_weight, Wg, Wu, Wd, sWg, sWu, sWd):
    grid = (E + 1, NTB)
    return pl.pallas_call(
        _moe_kernel,
        grid=grid,
        in_specs=[
            pl.BlockSpec((TB, D), lambda e, tb: (tb, 0)),
            pl.BlockSpec((E, D), lambda e, tb: (0, 0)),
            pl.BlockSpec((1, DFF, D), lambda e, tb: (jnp.minimum(e, E - 1), 0, 0)),
            pl.BlockSpec((1, DFF, D), lambda e, tb: (jnp.minimum(e, E - 1), 0, 0)),
            pl.BlockSpec((1, D, DFF), lambda e, tb: (jnp.minimum(e, E - 1), 0, 0)),
            pl.BlockSpec((SHARED_DFF, D), lambda e, tb: (0, 0)),
            pl.BlockSpec((SHARED_DFF, D), lambda e, tb: (0, 0)),
            pl.BlockSpec((D, SHARED_DFF), lambda e, tb: (0, 0)),
        ],
        out_specs=pl.BlockSpec((TB, D), lambda e, tb: (tb, 0)),
        out_shape=jax.ShapeDtypeStruct((T, D), jnp.float32),
        scratch_shapes=[pltpu.VMEM((T, D), jnp.float32),
                        pltpu.VMEM((T, E), jnp.float32)],
        compiler_params=pltpu.CompilerParams(
            dimension_semantics=("arbitrary", "arbitrary")),
    )(hidden_states, gate_weight, Wg, Wu, Wd, sWg, sWu, sWd)
